# Initial kernel scaffold; baseline (speedup 1.0000x reference)
import jax
import jax.numpy as jnp
from jax.experimental import pallas as pl


def kernel(x, edge_index, edge_weights, temporal_positions, W_in, b_in, W_t, b_t, Wq0, bq0, Wk0, bk0, Wv0, bv0, attn0, g0, be0, Wq1, bq1, Wk1, bk1, Wv1, bv1, attn1, g1, be1, W_qkv, b_qkv, W_o, b_o, W_out, b_out, g_f, b_f):
    raise NotImplementedError("write your pallas kernel here")


# speedup vs baseline: 9.0111x; 9.0111x over previous
import functools

import jax
import jax.numpy as jnp
from jax import lax
from jax.experimental import pallas as pl
from jax.experimental.pallas import tpu as pltpu

N = 2048
D = 256
E = 65536
H = 4
HD = D // H
BLK = 128
GRID = N // BLK


def _ln(x, g, b):
    m = jnp.mean(x, axis=-1, keepdims=True)
    v = jnp.mean((x - m) ** 2, axis=-1, keepdims=True)
    return (x - m) * jax.lax.rsqrt(v + 1e-5) * g + b


def _fold_attn(Wq, bq, Wk, bk, attn):
    Wq3 = Wq.reshape(H, HD, D)
    Wk3 = Wk.reshape(H, HD, D)
    wq_cols = jnp.einsum('hkd,hk->dh', Wq3, attn[:, :HD])
    wk_cols = jnp.einsum('hkd,hk->dh', Wk3, attn[:, HD:])
    Wsc = jnp.concatenate([wq_cols, wk_cols], axis=1)
    bq_off = jnp.sum(bq.reshape(H, HD) * attn[:, :HD], axis=1)
    bk_off = jnp.sum(bk.reshape(H, HD) * attn[:, HD:], axis=1)
    bsc = jnp.concatenate([bq_off, bk_off])
    return Wsc, bsc


def _proj_next(h_blk, Wv, bv, Wsc, bsc):
    v_blk = jnp.dot(h_blk, Wv.T, preferred_element_type=jnp.float32) + bv[None, :]
    sc_blk = jnp.dot(h_blk, Wsc, preferred_element_type=jnp.float32) + bsc[None, :]
    return v_blk, sc_blk


def _k1_body(x_ref, tp_ref, W_in_ref, b_in_ref, W_t_ref, b_t_ref,
             Wv_ref, bv_ref, Wsc_ref, bsc_ref,
             h_out, v_out, sc_out):
    x = x_ref[...]
    h0 = jnp.dot(x, W_in_ref[...].T, preferred_element_type=jnp.float32)
    h0 = h0 + b_in_ref[...][None, :]
    h0 = h0 + tp_ref[...].reshape(BLK, 1) * W_t_ref[...][:, 0][None, :]
    h0 = h0 + b_t_ref[...][None, :]
    h_out[...] = h0
    v_blk, sc_blk = _proj_next(h0, Wv_ref[...], bv_ref[...],
                               Wsc_ref[...], bsc_ref[...])
    v_out[...] = v_blk
    sc_out[...] = sc_blk


def _row_spec(width):
    return pl.BlockSpec((BLK, width), lambda i: (i, 0))


def _full_spec(shape):
    return pl.BlockSpec(shape, lambda i: tuple(0 for _ in shape))


def _vec_spec():
    return pl.BlockSpec((BLK,), lambda i: (i,))


def _k1(x, tp, W_in, b_in, W_t, b_t, Wv, bv, Wsc, bsc):
    return pl.pallas_call(
        _k1_body,
        grid=(GRID,),
        in_specs=[_row_spec(D), _vec_spec(), _full_spec((D, D)),
                  _full_spec((D,)), _full_spec((D, 1)), _full_spec((D,)),
                  _full_spec((D, D)), _full_spec((D,)),
                  _full_spec((D, 2 * H)), _full_spec((2 * H,))],
        out_specs=[_row_spec(D), _row_spec(D), _row_spec(2 * H)],
        out_shape=[jax.ShapeDtypeStruct((N, D), jnp.float32),
                   jax.ShapeDtypeStruct((N, D), jnp.float32),
                   jax.ShapeDtypeStruct((N, 2 * H), jnp.float32)],
    )(x, tp, W_in, b_in, W_t, b_t, Wv, bv, Wsc, bsc)


def _agg_epilogue(S_blk, M_extra, base_blk, V_full, h_blk, g, b):
    M = jnp.dot(S_blk, V_full, preferred_element_type=jnp.float32)
    if M_extra is not None:
        M = M + M_extra
    vsum = jnp.sum(V_full, axis=0)
    msgs = M + base_blk.reshape(BLK, 1) * vsum[None, :]
    return _ln(msgs + h_blk, g, b)


def _k2_body(S_ref, base_ref, V_ref, h_ref, g_ref, b_ref,
             Wv_ref, bv_ref, Wsc_ref, bsc_ref,
             h_out, v_out, sc_out):
    h1 = _agg_epilogue(S_ref[...], None, base_ref[...], V_ref[...],
                       h_ref[...], g_ref[...], b_ref[...])
    h_out[...] = h1
    v_blk, sc_blk = _proj_next(h1, Wv_ref[...], bv_ref[...],
                               Wsc_ref[...], bsc_ref[...])
    v_out[...] = v_blk
    sc_out[...] = sc_blk


def _k2(S, base, V, h, g, b, Wv, bv, Wsc, bsc):
    return pl.pallas_call(
        _k2_body,
        grid=(GRID,),
        in_specs=[_row_spec(N), _vec_spec(), _full_spec((N, D)),
                  _row_spec(D), _full_spec((D,)), _full_spec((D,)),
                  _full_spec((D, D)), _full_spec((D,)),
                  _full_spec((D, 2 * H)), _full_spec((2 * H,))],
        out_specs=[_row_spec(D), _row_spec(D), _row_spec(2 * H)],
        out_shape=[jax.ShapeDtypeStruct((N, D), jnp.float32),
                   jax.ShapeDtypeStruct((N, D), jnp.float32),
                   jax.ShapeDtypeStruct((N, 2 * H), jnp.float32)],
    )(S, base, V, h, g, b, Wv, bv, Wsc, bsc)


def _k3_body(S_ref, base_ref, V_ref, h_ref, g_ref, b_ref,
             Wqkv_ref, bqkv_ref, h_out, qkv_out):
    h2 = _agg_epilogue(S_ref[...], None, base_ref[...], V_ref[...],
                       h_ref[...], g_ref[...], b_ref[...])
    h_out[...] = h2
    qkv_out[...] = jnp.dot(h2, Wqkv_ref[...].T,
                           preferred_element_type=jnp.float32) + bqkv_ref[...][None, :]


def _k3(S, base, V, h, g, b, Wqkv, bqkv):
    return pl.pallas_call(
        _k3_body,
        grid=(GRID,),
        in_specs=[_row_spec(N), _vec_spec(), _full_spec((N, D)),
                  _row_spec(D), _full_spec((D,)), _full_spec((D,)),
                  _full_spec((3 * D, D)), _full_spec((3 * D,))],
        out_specs=[_row_spec(D), _row_spec(3 * D)],
        out_shape=[jax.ShapeDtypeStruct((N, D), jnp.float32),
                   jax.ShapeDtypeStruct((N, 3 * D), jnp.float32)],
    )(S, base, V, h, g, b, Wqkv, bqkv)


def _k5(qkv, Wo, bo, Wout, bout, gf, bf):
    def body(qkv_blk_ref, qkv_full_ref, Wo_ref, bo_ref, Wout_ref, bout_ref,
             gf_ref, bf_ref, out_ref):
        qb = qkv_blk_ref[...][:, :D]
        k_full = qkv_full_ref[...][:, D:2 * D]
        v_full = qkv_full_ref[...][:, 2 * D:]
        outs = []
        for hh in range(H):
            qh = qb[:, hh * HD:(hh + 1) * HD]
            kh = k_full[:, hh * HD:(hh + 1) * HD]
            vh = v_full[:, hh * HD:(hh + 1) * HD]
            s = jnp.dot(qh, kh.T, preferred_element_type=jnp.float32) / 8.0
            m = jnp.max(s, axis=-1, keepdims=True)
            e = jnp.exp(s - m)
            p = e / jnp.sum(e, axis=-1, keepdims=True)
            outs.append(jnp.dot(p, vh, preferred_element_type=jnp.float32))
        o = jnp.concatenate(outs, axis=-1)
        o = jnp.dot(o, Wo_ref[...].T, preferred_element_type=jnp.float32) + bo_ref[...][None, :]
        o = jnp.dot(o, Wout_ref[...].T, preferred_element_type=jnp.float32) + bout_ref[...][None, :]
        out_ref[...] = _ln(o, gf_ref[...], bf_ref[...])

    return pl.pallas_call(
        body,
        grid=(GRID,),
        in_specs=[_row_spec(3 * D), _full_spec((N, 3 * D)),
                  _full_spec((D, D)), _full_spec((D,)),
                  _full_spec((D, D)), _full_spec((D,)),
                  _full_spec((D,)), _full_spec((D,))],
        out_specs=_row_spec(D),
        out_shape=jax.ShapeDtypeStruct((N, D), jnp.float32),
    )(qkv, qkv, Wo, bo, Wout, bout, gf, bf)


def _edge_pass(scores, ew, src, dst):
    qs = scores[:, :H]
    ks = scores[:, H:]
    s = qs[src] + ks[dst]
    s = jnp.where(s > 0, s, 0.2 * s) * ew[:, None]
    t = jnp.exp(s) - 1.0
    Z = float(N) + jnp.zeros((N, H), jnp.float32).at[dst].add(t)
    Zinv = 1.0 / Z
    c = jnp.sum(t * Zinv[dst], axis=-1)
    S = jnp.zeros((N, N), jnp.float32).at[dst, src].add(c)
    base = jnp.sum(Zinv, axis=-1)
    return S, base


def kernel(x, edge_index, edge_weights, temporal_positions, W_in, b_in, W_t,
           b_t, Wq0, bq0, Wk0, bk0, Wv0, bv0, attn0, g0, be0, Wq1, bq1, Wk1,
           bk1, Wv1, bv1, attn1, g1, be1, W_qkv, b_qkv, W_o, b_o, W_out,
           b_out, g_f, b_f):
    src = edge_index[0]
    dst = edge_index[1]
    Wsc0, bsc0 = _fold_attn(Wq0, bq0, Wk0, bk0, attn0)
    Wsc1, bsc1 = _fold_attn(Wq1, bq1, Wk1, bk1, attn1)

    h0, V0, sc0 = _k1(x, temporal_positions, W_in, b_in, W_t, b_t,
                      Wv0, bv0, Wsc0, bsc0)
    S0, base0 = _edge_pass(sc0, edge_weights, src, dst)
    h1, V1, sc1 = _k2(S0, base0, V0, h0, g0, be0,
                      Wv1, bv1, Wsc1, bsc1)
    S1, base1 = _edge_pass(sc1, edge_weights, src, dst)
    h2, qkv = _k3(S1, base1, V1, h1, g1, be1, W_qkv, b_qkv)
    return _k5(qkv, W_o, b_o, W_out, b_out, g_f, b_f)
